# baseline (speedup 1.0000x reference)
import jax
import jax.numpy as jnp
from jax.experimental import pallas as pl
from jax.experimental.pallas import tpu as pltpu

B, D, H, R, K, NL = 4096, 384, 64, 8, 2, 64
RH = R * H
AUX_COEF = 0.05
TILE = 1024
GRID = B // TILE

_NEG = -3.0e38


def _dot(a, b):
    return jnp.dot(a, b, preferred_element_type=jnp.float32)


def _moe_kernel(x1_ref, x2_ref, leT_ref, gw_ref, gbc_ref, uc_ref, ub_ref,
                vc_ref, vbc_ref, g_ref, fz_ref, ebc_ref, tri_ref, f_ref,
                ones_ref, rand_ref, sel_ref, logp_ref, aux_ref,
                m_ref, accp_ref, accm_ref):
    i = pl.program_id(0)
    x1 = x1_ref[...]
    x2 = x2_ref[...]

    @pl.when(i == 0)
    def _():
        g = g_ref[...]
        eht = jax.lax.dot_general(vc_ref[...], leT_ref[...],
                                  (((0,), (0,)), ((), ())),
                                  preferred_element_type=jnp.float32)
        eht = eht + vbc_ref[...]
        en2 = _dot(g, eht * eht)
        ehn = eht / jnp.maximum(jnp.sqrt(en2), 1e-12)
        m_ref[...] = jnp.concatenate([ehn] * R, axis=1) * g
        accp_ref[...] = jnp.zeros_like(accp_ref)
        accm_ref[...] = jnp.zeros_like(accm_ref)

    gw = gw_ref[...]
    lgT = (jax.lax.dot_general(gw[:, :D], x1, (((1,), (1,)), ((), ())),
                               preferred_element_type=jnp.float32)
           + jax.lax.dot_general(gw[:, D:], x2, (((1,), (1,)), ((), ())),
                                 preferred_element_type=jnp.float32)
           + gbc_ref[...])

    riT = jax.lax.broadcasted_iota(jnp.int32, lgT.shape, 0)
    m1 = jnp.max(lgT, axis=0, keepdims=True)
    i1 = jnp.min(jnp.where(lgT == m1, riT, R), axis=0, keepdims=True)
    lgm = jnp.where(riT == i1, _NEG, lgT)
    m2 = jnp.max(lgm, axis=0, keepdims=True)
    i2 = jnp.min(jnp.where(lgm == m2, riT, R), axis=0, keepdims=True)

    e2 = jnp.exp(m2 - m1)
    w1 = 1.0 / (1.0 + e2)
    w2 = e2 / (1.0 + e2)

    p = jnp.exp(lgT - m1)
    probsT = p / jnp.sum(p, axis=0, keepdims=True)
    maskT = ((riT == i1) | (riT == i2)).astype(jnp.float32)
    accp_ref[...] += jnp.sum(probsT, axis=1, keepdims=True)
    accm_ref[...] += jnp.sum(maskT, axis=1, keepdims=True)
    aux_ref[...] = (R * AUX_COEF / (B * B)) * jnp.sum(
        accp_ref[...] * accm_ref[...], axis=(0, 1), keepdims=True)

    w8T = (jnp.where(riT == i1, w1, 0.0)
           + jnp.where(riT == i2, w2, 0.0))
    w8 = w8T.T

    xh = _dot(x1, uc_ref[:D]) + _dot(x2, uc_ref[D:]) + ub_ref[...]
    n2 = _dot(xh * xh, g_ref[...])
    xhn = xh / jnp.maximum(jnp.sqrt(n2), 1e-12)

    s = _dot(xhn, m_ref[...])
    es = jnp.exp(s)
    z = _dot(es, g_ref[...])
    pr = es / z
    wl = _dot(w8, ebc_ref[...])
    llm_probs = _dot(pr * wl, f_ref[...])

    csum = _dot(llm_probs, tri_ref[...])
    rand = rand_ref[...]
    cf = _dot((csum <= rand).astype(jnp.float32), ones_ref[...])
    cnt = cf.astype(jnp.int32)
    sel = jnp.where(cnt == NL, 0, cnt)
    sel_ref[...] = sel

    n_iota = jax.lax.broadcasted_iota(jnp.int32, llm_probs.shape, 1)
    psel = _dot(jnp.where(n_iota == sel, llm_probs, 0.0), ones_ref[...])
    logp_ref[...] = jnp.log(psel)


@jax.jit
def kernel(enhanced_posts_embeddings, selected_reasoning_embeddings,
           llm_embeddings, gate_W, gate_b, U_W, U_b, V_W, V_b):
    uc = U_W.transpose(2, 0, 1).reshape(2 * D, RH)
    ub = U_b.reshape(1, RH)
    vc = V_W.transpose(2, 0, 1).reshape(D, RH)
    vbc = V_b.reshape(RH, 1)
    gbc = gate_b.reshape(R, 1)
    leT = llm_embeddings.T
    rand = jax.random.uniform(jax.random.key(42), (B, 1))

    ri = jnp.arange(RH)
    g_blk = (ri[:, None] // H == ri[None, :] // H).astype(jnp.float32)
    fz = (ri[:, None] // H == jnp.arange(R)[None, :]).astype(jnp.float32)
    ebc = fz.T
    nn = jnp.arange(NL)
    tri = (nn[:, None] <= nn[None, :]).astype(jnp.float32)
    f_fold = (ri[:, None] % NL == nn[None, :]).astype(jnp.float32)
    ones_col = jnp.ones((NL, 1), jnp.float32)

    cspec = lambda shape: pl.BlockSpec(shape, lambda i: (0,) * len(shape))
    sel, logp, aux = pl.pallas_call(
        _moe_kernel,
        grid=(GRID,),
        in_specs=[
            pl.BlockSpec((TILE, D), lambda i: (i, 0)),
            pl.BlockSpec((TILE, D), lambda i: (i, 0)),
            cspec((D, NL)),
            cspec((R, 2 * D)),
            cspec((R, 1)),
            cspec((2 * D, RH)),
            cspec((1, RH)),
            cspec((D, RH)),
            cspec((RH, 1)),
            cspec((RH, RH)),
            cspec((RH, R)),
            cspec((R, RH)),
            cspec((NL, NL)),
            cspec((RH, NL)),
            cspec((NL, 1)),
            pl.BlockSpec((TILE, 1), lambda i: (i, 0)),
        ],
        out_specs=[
            pl.BlockSpec((TILE, 1), lambda i: (i, 0)),
            pl.BlockSpec((TILE, 1), lambda i: (i, 0)),
            pl.BlockSpec((1, 1), lambda i: (0, 0)),
        ],
        out_shape=[
            jax.ShapeDtypeStruct((B, 1), jnp.int32),
            jax.ShapeDtypeStruct((B, 1), jnp.float32),
            jax.ShapeDtypeStruct((1, 1), jnp.float32),
        ],
        scratch_shapes=[pltpu.VMEM((RH, RH), jnp.float32),
                        pltpu.VMEM((R, 1), jnp.float32),
                        pltpu.VMEM((R, 1), jnp.float32)],
    )(enhanced_posts_embeddings, selected_reasoning_embeddings,
      leT, gate_W, gbc, uc, ub, vc, vbc, g_blk, fz, ebc, tri, f_fold,
      ones_col, rand)
    return sel[:, 0], logp, aux[0, 0]


# speedup vs baseline: 1.1399x; 1.1399x over previous
import jax
import jax.numpy as jnp
from jax.experimental import pallas as pl
from jax.experimental.pallas import tpu as pltpu

B, D, H, R, K, NL = 4096, 384, 64, 8, 2, 64
RH = R * H
AUX_COEF = 0.05
TILE = 1024
GRID = B // TILE

_NEG = -3.0e38


def _dot(a, b):
    return jnp.dot(a, b, preferred_element_type=jnp.float32)


def _dot_t(a, b):
    return jax.lax.dot_general(a, b, (((1,), (1,)), ((), ())),
                               preferred_element_type=jnp.float32)


def _moe_kernel(x1_ref, x2_ref, le_ref, gw_ref, gbc_ref, uw_ref, ub_ref,
                vw_ref, vbc_ref, g_ref, ebc_ref, tri_ref, f_ref,
                ones_ref, rand_ref, sel_ref, logp_ref, aux_ref,
                m_ref, accp_ref, accm_ref):
    i = pl.program_id(0)
    x1 = x1_ref[...]
    x2 = x2_ref[...]

    @pl.when(i == 0)
    def _():
        g = g_ref[...]
        eht = _dot_t(vw_ref[...], le_ref[...]) + vbc_ref[...]
        en2 = _dot(g, eht * eht)
        ehn = eht / jnp.maximum(jnp.sqrt(en2), 1e-12)
        m_ref[...] = jnp.concatenate([ehn] * R, axis=1) * g
        accp_ref[...] = jnp.zeros_like(accp_ref)
        accm_ref[...] = jnp.zeros_like(accm_ref)

    gw = gw_ref[...]
    lgT = (_dot_t(gw[:, :D], x1) + _dot_t(gw[:, D:], x2)
           + gbc_ref[...])

    riT = jax.lax.broadcasted_iota(jnp.int32, lgT.shape, 0)
    m1 = jnp.max(lgT, axis=0, keepdims=True)
    i1 = jnp.min(jnp.where(lgT == m1, riT, R), axis=0, keepdims=True)
    lgm = jnp.where(riT == i1, _NEG, lgT)
    m2 = jnp.max(lgm, axis=0, keepdims=True)
    i2 = jnp.min(jnp.where(lgm == m2, riT, R), axis=0, keepdims=True)

    e2 = jnp.exp(m2 - m1)
    w1 = 1.0 / (1.0 + e2)
    w2 = e2 / (1.0 + e2)

    p = jnp.exp(lgT - m1)
    probsT = p / jnp.sum(p, axis=0, keepdims=True)
    maskT = ((riT == i1) | (riT == i2)).astype(jnp.float32)
    accp_ref[...] += jnp.sum(probsT, axis=1, keepdims=True)
    accm_ref[...] += jnp.sum(maskT, axis=1, keepdims=True)
    aux_ref[...] = (R * AUX_COEF / (B * B)) * jnp.sum(
        accp_ref[...] * accm_ref[...], axis=(0, 1), keepdims=True)

    w8T = (jnp.where(riT == i1, w1, 0.0)
           + jnp.where(riT == i2, w2, 0.0))
    w8 = w8T.T

    uw = uw_ref[...]
    xh = _dot_t(x1, uw[:, :D]) + _dot_t(x2, uw[:, D:]) + ub_ref[...]
    n2 = _dot(xh * xh, g_ref[...])
    xhn = xh / jnp.maximum(jnp.sqrt(n2), 1e-12)

    s = _dot(xhn, m_ref[...])
    es = jnp.exp(s)
    z = _dot(es, g_ref[...])
    pr = es / z
    wl = _dot(w8, ebc_ref[...])
    llm_probs = _dot(pr * wl, f_ref[...])

    csum = _dot(llm_probs, tri_ref[...])
    rand = rand_ref[...]
    cf = _dot((csum <= rand).astype(jnp.float32), ones_ref[...])
    cnt = cf.astype(jnp.int32)
    sel = jnp.where(cnt == NL, 0, cnt)
    sel_ref[...] = sel

    n_iota = jax.lax.broadcasted_iota(jnp.int32, llm_probs.shape, 1)
    psel = _dot(jnp.where(n_iota == sel, llm_probs, 0.0), ones_ref[...])
    logp_ref[...] = jnp.log(psel)


@jax.jit
def kernel(enhanced_posts_embeddings, selected_reasoning_embeddings,
           llm_embeddings, gate_W, gate_b, U_W, U_b, V_W, V_b):
    uw = U_W.reshape(RH, 2 * D)
    ub = U_b.reshape(1, RH)
    vw = V_W.reshape(RH, D)
    vbc = V_b.reshape(RH, 1)
    gbc = gate_b.reshape(R, 1)
    rand = jax.random.uniform(jax.random.key(42), (B, 1))

    ri = jnp.arange(RH)
    g_blk = (ri[:, None] // H == ri[None, :] // H).astype(jnp.float32)
    ebc = (jnp.arange(R)[:, None] == ri[None, :] // H).astype(jnp.float32)
    nn = jnp.arange(NL)
    tri = (nn[:, None] <= nn[None, :]).astype(jnp.float32)
    f_fold = (ri[:, None] % NL == nn[None, :]).astype(jnp.float32)
    ones_col = jnp.ones((NL, 1), jnp.float32)

    cspec = lambda shape: pl.BlockSpec(shape, lambda i: (0,) * len(shape))
    sel, logp, aux = pl.pallas_call(
        _moe_kernel,
        grid=(GRID,),
        in_specs=[
            pl.BlockSpec((TILE, D), lambda i: (i, 0)),
            pl.BlockSpec((TILE, D), lambda i: (i, 0)),
            cspec((NL, D)),
            cspec((R, 2 * D)),
            cspec((R, 1)),
            cspec((RH, 2 * D)),
            cspec((1, RH)),
            cspec((RH, D)),
            cspec((RH, 1)),
            cspec((RH, RH)),
            cspec((R, RH)),
            cspec((NL, NL)),
            cspec((RH, NL)),
            cspec((NL, 1)),
            pl.BlockSpec((TILE, 1), lambda i: (i, 0)),
        ],
        out_specs=[
            pl.BlockSpec((TILE, 1), lambda i: (i, 0)),
            pl.BlockSpec((TILE, 1), lambda i: (i, 0)),
            pl.BlockSpec((1, 1), lambda i: (0, 0)),
        ],
        out_shape=[
            jax.ShapeDtypeStruct((B, 1), jnp.int32),
            jax.ShapeDtypeStruct((B, 1), jnp.float32),
            jax.ShapeDtypeStruct((1, 1), jnp.float32),
        ],
        scratch_shapes=[pltpu.VMEM((RH, RH), jnp.float32),
                        pltpu.VMEM((R, 1), jnp.float32),
                        pltpu.VMEM((R, 1), jnp.float32)],
    )(enhanced_posts_embeddings, selected_reasoning_embeddings,
      llm_embeddings, gate_W, gbc, uw, ub, vw, vbc, g_blk, ebc, tri, f_fold,
      ones_col, rand)
    return sel[:, 0], logp, aux[0, 0]


# speedup vs baseline: 1.1498x; 1.0086x over previous
import jax
import jax.numpy as jnp
from jax.experimental import pallas as pl
from jax.experimental.pallas import tpu as pltpu

B, D, H, R, K, NL = 4096, 384, 64, 8, 2, 64
RH = R * H
AUX_COEF = 0.05
TILE = 1024
GRID = B // TILE

_NEG = -3.0e38


def _dot(a, b):
    return jnp.dot(a, b, preferred_element_type=jnp.float32)


def _dot_t(a, b):
    return jax.lax.dot_general(a, b, (((1,), (1,)), ((), ())),
                               preferred_element_type=jnp.float32)


def _moe_kernel(x1_ref, x2_ref, le_ref, gwt_ref, gb_ref, uw_ref, ub_ref,
                vw_ref, vbc_ref, g_ref, tri_ref, f_ref,
                ones_ref, rand_ref, sel_ref, logp_ref, aux_ref,
                m_ref, uc_ref, accp_ref, accm_ref):
    i = pl.program_id(0)
    x1 = x1_ref[...]
    x2 = x2_ref[...]

    @pl.when(i == 0)
    def _():
        g = g_ref[...]
        uc_ref[...] = uw_ref[...].T
        vc = vw_ref[...].T
        leT = le_ref[...].T
        eht = jax.lax.dot_general(vc, leT, (((0,), (0,)), ((), ())),
                                  preferred_element_type=jnp.float32)
        eht = eht + vbc_ref[...]
        en2 = _dot(g, eht * eht)
        ehn = eht / jnp.maximum(jnp.sqrt(en2), 1e-12)
        m_ref[...] = jnp.concatenate([ehn] * R, axis=1) * g
        accp_ref[...] = jnp.zeros_like(accp_ref)
        accm_ref[...] = jnp.zeros_like(accm_ref)

    gwt = gwt_ref[...]
    logits = (_dot(x1, gwt[:D]) + _dot(x2, gwt[D:])
              + gb_ref[...])

    r_iota = jax.lax.broadcasted_iota(jnp.int32, logits.shape, 1)
    m1 = jnp.max(logits, axis=1, keepdims=True)
    i1 = jnp.min(jnp.where(logits == m1, r_iota, R), axis=1, keepdims=True)
    lgm = jnp.where(r_iota == i1, _NEG, logits)
    m2 = jnp.max(lgm, axis=1, keepdims=True)
    i2 = jnp.min(jnp.where(lgm == m2, r_iota, R), axis=1, keepdims=True)

    e2 = jnp.exp(m2 - m1)
    w1 = 1.0 / (1.0 + e2)
    w2 = e2 / (1.0 + e2)

    p = jnp.exp(logits - m1)
    probs = p / jnp.sum(p, axis=1, keepdims=True)
    mask = ((r_iota == i1) | (r_iota == i2)).astype(jnp.float32)
    accp_ref[...] += jnp.sum(probs, axis=0, keepdims=True)
    accm_ref[...] += jnp.sum(mask, axis=0, keepdims=True)
    aux_ref[...] = (R * AUX_COEF / (B * B)) * jnp.sum(
        accp_ref[...] * accm_ref[...], axis=1, keepdims=True)

    xh = _dot(x1, uc_ref[:D]) + _dot(x2, uc_ref[D:]) + ub_ref[...]
    n2 = _dot(xh * xh, g_ref[...])
    xhn = xh / jnp.maximum(jnp.sqrt(n2), 1e-12)

    s = _dot(xhn, m_ref[...])
    es = jnp.exp(s)
    z = _dot(es, g_ref[...])
    pr = es / z

    lane_r = jax.lax.broadcasted_iota(jnp.int32, pr.shape, 1) // NL
    w = jnp.where(lane_r == i1, w1, 0.0) + jnp.where(lane_r == i2, w2, 0.0)
    llm_probs = _dot(pr * w, f_ref[...])

    csum = _dot(llm_probs, tri_ref[...])
    rand = rand_ref[...]
    cf = _dot((csum <= rand).astype(jnp.float32), ones_ref[...])
    cnt = cf.astype(jnp.int32)
    sel = jnp.where(cnt == NL, 0, cnt)
    sel_ref[...] = sel

    n_iota = jax.lax.broadcasted_iota(jnp.int32, llm_probs.shape, 1)
    psel = _dot(jnp.where(n_iota == sel, llm_probs, 0.0), ones_ref[...])
    logp_ref[...] = jnp.log(psel)


@jax.jit
def kernel(enhanced_posts_embeddings, selected_reasoning_embeddings,
           llm_embeddings, gate_W, gate_b, U_W, U_b, V_W, V_b):
    uw = U_W.reshape(RH, 2 * D)
    ub = U_b.reshape(1, RH)
    vw = V_W.reshape(RH, D)
    vbc = V_b.reshape(RH, 1)
    gwt = gate_W.T
    gb2 = gate_b.reshape(1, R)
    rand = jax.random.uniform(jax.random.key(42), (B, 1))

    ri = jnp.arange(RH)
    g_blk = (ri[:, None] // H == ri[None, :] // H).astype(jnp.float32)
    nn = jnp.arange(NL)
    tri = (nn[:, None] <= nn[None, :]).astype(jnp.float32)
    f_fold = (ri[:, None] % NL == nn[None, :]).astype(jnp.float32)
    ones_col = jnp.ones((NL, 1), jnp.float32)

    cspec = lambda shape: pl.BlockSpec(shape, lambda i: (0,) * len(shape))
    sel, logp, aux = pl.pallas_call(
        _moe_kernel,
        grid=(GRID,),
        in_specs=[
            pl.BlockSpec((TILE, D), lambda i: (i, 0)),
            pl.BlockSpec((TILE, D), lambda i: (i, 0)),
            cspec((NL, D)),
            cspec((2 * D, R)),
            cspec((1, R)),
            cspec((RH, 2 * D)),
            cspec((1, RH)),
            cspec((RH, D)),
            cspec((RH, 1)),
            cspec((RH, RH)),
            cspec((NL, NL)),
            cspec((RH, NL)),
            cspec((NL, 1)),
            pl.BlockSpec((TILE, 1), lambda i: (i, 0)),
        ],
        out_specs=[
            pl.BlockSpec((TILE, 1), lambda i: (i, 0)),
            pl.BlockSpec((TILE, 1), lambda i: (i, 0)),
            pl.BlockSpec((1, 1), lambda i: (0, 0)),
        ],
        out_shape=[
            jax.ShapeDtypeStruct((B, 1), jnp.int32),
            jax.ShapeDtypeStruct((B, 1), jnp.float32),
            jax.ShapeDtypeStruct((1, 1), jnp.float32),
        ],
        scratch_shapes=[pltpu.VMEM((RH, RH), jnp.float32),
                        pltpu.VMEM((2 * D, RH), jnp.float32),
                        pltpu.VMEM((1, R), jnp.float32),
                        pltpu.VMEM((1, R), jnp.float32)],
    )(enhanced_posts_embeddings, selected_reasoning_embeddings,
      llm_embeddings, gwt, gb2, uw, ub, vw, vbc, g_blk, tri, f_fold,
      ones_col, rand)
    return sel[:, 0], logp, aux[0, 0]


# speedup vs baseline: 1.5172x; 1.3196x over previous
import jax
import jax.numpy as jnp
import numpy as np
from jax.experimental import pallas as pl
from jax.experimental.pallas import tpu as pltpu

B, D, H, R, K, NL = 4096, 384, 64, 8, 2, 64
RH = R * H
AUX_COEF = 0.05
TILE = 1024
GRID = B // TILE

_NEG = -3.0e38

_RAND = np.asarray(jax.random.uniform(jax.random.key(42), (B, 1),
                                      jnp.float32))
_RI = np.arange(RH)
_G_BLK = (_RI[:, None] // H == _RI[None, :] // H).astype(np.float32)
_NN = np.arange(NL)
_TRI = (_NN[:, None] <= _NN[None, :]).astype(np.float32)
_F_FOLD = (_RI[:, None] % NL == _NN[None, :]).astype(np.float32)
_ONES_COL = np.ones((NL, 1), np.float32)


def _dot(a, b):
    return jnp.dot(a, b, preferred_element_type=jnp.float32)


def _dot_t(a, b):
    return jax.lax.dot_general(a, b, (((1,), (1,)), ((), ())),
                               preferred_element_type=jnp.float32)


def _moe_kernel(x1_ref, x2_ref, le_ref, gwt_ref, gb_ref, uw_ref, ub_ref,
                vw_ref, vbc_ref, g_ref, tri_ref, f_ref,
                ones_ref, rand_ref, sel_ref, logp_ref, aux_ref,
                m_ref, uc_ref, accp_ref, accm_ref):
    i = pl.program_id(0)
    x1 = x1_ref[...]
    x2 = x2_ref[...]

    @pl.when(i == 0)
    def _():
        g = g_ref[...]
        uc_ref[...] = uw_ref[...].T
        vc = vw_ref[...].T
        leT = le_ref[...].T
        eht = jax.lax.dot_general(vc, leT, (((0,), (0,)), ((), ())),
                                  preferred_element_type=jnp.float32)
        eht = eht + vbc_ref[...]
        en2 = _dot(g, eht * eht)
        ehn = eht / jnp.maximum(jnp.sqrt(en2), 1e-12)
        m_ref[...] = jnp.concatenate([ehn] * R, axis=1) * g
        accp_ref[...] = jnp.zeros_like(accp_ref)
        accm_ref[...] = jnp.zeros_like(accm_ref)

    gwt = gwt_ref[...]
    logits = (_dot(x1, gwt[:D]) + _dot(x2, gwt[D:])
              + gb_ref[...])

    r_iota = jax.lax.broadcasted_iota(jnp.int32, logits.shape, 1)
    m1 = jnp.max(logits, axis=1, keepdims=True)
    i1 = jnp.min(jnp.where(logits == m1, r_iota, R), axis=1, keepdims=True)
    lgm = jnp.where(r_iota == i1, _NEG, logits)
    m2 = jnp.max(lgm, axis=1, keepdims=True)
    i2 = jnp.min(jnp.where(lgm == m2, r_iota, R), axis=1, keepdims=True)

    e2 = jnp.exp(m2 - m1)
    w1 = 1.0 / (1.0 + e2)
    w2 = e2 / (1.0 + e2)

    p = jnp.exp(logits - m1)
    probs = p / jnp.sum(p, axis=1, keepdims=True)
    mask = ((r_iota == i1) | (r_iota == i2)).astype(jnp.float32)
    accp_ref[...] += jnp.sum(probs, axis=0, keepdims=True)
    accm_ref[...] += jnp.sum(mask, axis=0, keepdims=True)
    aux_ref[...] = (R * AUX_COEF / (B * B)) * jnp.sum(
        accp_ref[...] * accm_ref[...], axis=1, keepdims=True)

    xh = _dot(x1, uc_ref[:D]) + _dot(x2, uc_ref[D:]) + ub_ref[...]
    n2 = _dot(xh * xh, g_ref[...])
    xhn = xh / jnp.maximum(jnp.sqrt(n2), 1e-12)

    s = _dot(xhn, m_ref[...])
    es = jnp.exp(s)
    z = _dot(es, g_ref[...])
    pr = es / z

    lane_r = jax.lax.broadcasted_iota(jnp.int32, pr.shape, 1) // NL
    w = jnp.where(lane_r == i1, w1, 0.0) + jnp.where(lane_r == i2, w2, 0.0)
    llm_probs = _dot(pr * w, f_ref[...])

    csum = _dot(llm_probs, tri_ref[...])
    rand = rand_ref[...]
    cf = _dot((csum <= rand).astype(jnp.float32), ones_ref[...])
    cnt = cf.astype(jnp.int32)
    sel = jnp.where(cnt == NL, 0, cnt)
    sel_ref[...] = sel

    n_iota = jax.lax.broadcasted_iota(jnp.int32, llm_probs.shape, 1)
    psel = _dot(jnp.where(n_iota == sel, llm_probs, 0.0), ones_ref[...])
    logp_ref[...] = jnp.log(psel)


@jax.jit
def kernel(enhanced_posts_embeddings, selected_reasoning_embeddings,
           llm_embeddings, gate_W, gate_b, U_W, U_b, V_W, V_b):
    uw = U_W.reshape(RH, 2 * D)
    ub = U_b.reshape(1, RH)
    vw = V_W.reshape(RH, D)
    vbc = V_b.reshape(RH, 1)
    gwt = gate_W.T
    gb2 = gate_b.reshape(1, R)
    rand = jnp.asarray(_RAND)
    g_blk = jnp.asarray(_G_BLK)
    tri = jnp.asarray(_TRI)
    f_fold = jnp.asarray(_F_FOLD)
    ones_col = jnp.asarray(_ONES_COL)

    cspec = lambda shape: pl.BlockSpec(shape, lambda i: (0,) * len(shape))
    sel, logp, aux = pl.pallas_call(
        _moe_kernel,
        grid=(GRID,),
        in_specs=[
            pl.BlockSpec((TILE, D), lambda i: (i, 0)),
            pl.BlockSpec((TILE, D), lambda i: (i, 0)),
            cspec((NL, D)),
            cspec((2 * D, R)),
            cspec((1, R)),
            cspec((RH, 2 * D)),
            cspec((1, RH)),
            cspec((RH, D)),
            cspec((RH, 1)),
            cspec((RH, RH)),
            cspec((NL, NL)),
            cspec((RH, NL)),
            cspec((NL, 1)),
            pl.BlockSpec((TILE, 1), lambda i: (i, 0)),
        ],
        out_specs=[
            pl.BlockSpec((TILE, 1), lambda i: (i, 0)),
            pl.BlockSpec((TILE, 1), lambda i: (i, 0)),
            pl.BlockSpec((1, 1), lambda i: (0, 0)),
        ],
        out_shape=[
            jax.ShapeDtypeStruct((B, 1), jnp.int32),
            jax.ShapeDtypeStruct((B, 1), jnp.float32),
            jax.ShapeDtypeStruct((1, 1), jnp.float32),
        ],
        scratch_shapes=[pltpu.VMEM((RH, RH), jnp.float32),
                        pltpu.VMEM((2 * D, RH), jnp.float32),
                        pltpu.VMEM((1, R), jnp.float32),
                        pltpu.VMEM((1, R), jnp.float32)],
    )(enhanced_posts_embeddings, selected_reasoning_embeddings,
      llm_embeddings, gwt, gb2, uw, ub, vw, vbc, g_blk, tri, f_fold,
      ones_col, rand)
    return sel[:, 0], logp, aux[0, 0]


# speedup vs baseline: 1.7086x; 1.1262x over previous
import jax
import jax.numpy as jnp
import numpy as np
from jax.experimental import pallas as pl
from jax.experimental.pallas import tpu as pltpu

B, D, H, R, K, NL = 4096, 384, 64, 8, 2, 64
RH = R * H
GH = RH // 2
AUX_COEF = 0.05
TILE = 1024
GRID = B // TILE

_NEG = -3.0e38

_RAND = np.asarray(jax.random.uniform(jax.random.key(42), (B, 1),
                                      jnp.float32))
_GI = np.arange(GH)
_G4 = (_GI[:, None] // H == _GI[None, :] // H).astype(np.float32)
_NN = np.arange(NL)
_TRI = (_NN[:, None] <= _NN[None, :]).astype(np.float32)
_F4 = (_GI[:, None] % NL == _NN[None, :]).astype(np.float32)
_ONES_COL = np.ones((NL, 1), np.float32)


def _dot(a, b):
    return jnp.dot(a, b, preferred_element_type=jnp.float32)


def _moe_kernel(x1_ref, x2_ref, le_ref, gw_ref, gb_ref, uw_ref, ub_ref,
                vw_ref, vbc_ref, g4_ref, tri_ref, f4_ref,
                ones_ref, rand_ref, sel_ref, logp_ref, aux_ref,
                ma_ref, mb_ref, uc_ref, gt_ref, accp_ref, accm_ref):
    i = pl.program_id(0)
    x1 = x1_ref[...]
    x2 = x2_ref[...]
    g4 = g4_ref[...]

    @pl.when(i == 0)
    def _():
        uc_ref[...] = uw_ref[...].T
        gt_ref[...] = gw_ref[...].T
        vc = vw_ref[...].T
        leT = le_ref[...].T
        eht = jax.lax.dot_general(vc, leT, (((0,), (0,)), ((), ())),
                                  preferred_element_type=jnp.float32)
        eht = eht + vbc_ref[...]
        eha, ehb = eht[:GH], eht[GH:]
        ehna = eha / jnp.maximum(jnp.sqrt(_dot(g4, eha * eha)), 1e-12)
        ehnb = ehb / jnp.maximum(jnp.sqrt(_dot(g4, ehb * ehb)), 1e-12)
        ma_ref[...] = jnp.concatenate([ehna] * 4, axis=1) * g4
        mb_ref[...] = jnp.concatenate([ehnb] * 4, axis=1) * g4
        accp_ref[...] = jnp.zeros_like(accp_ref)
        accm_ref[...] = jnp.zeros_like(accm_ref)

    logits = (_dot(x1, gt_ref[:D]) + _dot(x2, gt_ref[D:])
              + gb_ref[...])

    r_iota = jax.lax.broadcasted_iota(jnp.int32, logits.shape, 1)
    m1 = jnp.max(logits, axis=1, keepdims=True)
    i1 = jnp.min(jnp.where(logits == m1, r_iota, R), axis=1, keepdims=True)
    lgm = jnp.where(r_iota == i1, _NEG, logits)
    m2 = jnp.max(lgm, axis=1, keepdims=True)
    i2 = jnp.min(jnp.where(lgm == m2, r_iota, R), axis=1, keepdims=True)

    e2 = jnp.exp(m2 - m1)
    w1 = 1.0 / (1.0 + e2)
    w2 = e2 / (1.0 + e2)

    p = jnp.exp(logits - m1)
    probs = p / jnp.sum(p, axis=1, keepdims=True)
    mask = ((r_iota == i1) | (r_iota == i2)).astype(jnp.float32)
    accp_ref[...] += jnp.sum(probs, axis=0, keepdims=True)
    accm_ref[...] += jnp.sum(mask, axis=0, keepdims=True)
    aux_ref[...] = (R * AUX_COEF / (B * B)) * jnp.sum(
        accp_ref[...] * accm_ref[...], axis=1, keepdims=True)

    xh = _dot(x1, uc_ref[:D]) + _dot(x2, uc_ref[D:]) + ub_ref[...]
    xa, xb = xh[:, :GH], xh[:, GH:]
    xhna = xa / jnp.maximum(jnp.sqrt(_dot(xa * xa, g4)), 1e-12)
    xhnb = xb / jnp.maximum(jnp.sqrt(_dot(xb * xb, g4)), 1e-12)

    esa = jnp.exp(_dot(xhna, ma_ref[...]))
    esb = jnp.exp(_dot(xhnb, mb_ref[...]))
    pra = esa / _dot(esa, g4)
    prb = esb / _dot(esb, g4)

    lane4 = jax.lax.broadcasted_iota(jnp.int32, pra.shape, 1) // NL
    wa = jnp.where(lane4 == i1, w1, 0.0) + jnp.where(lane4 == i2, w2, 0.0)
    lb = lane4 + 4
    wb = jnp.where(lb == i1, w1, 0.0) + jnp.where(lb == i2, w2, 0.0)
    llm_probs = _dot(pra * wa, f4_ref[...]) + _dot(prb * wb, f4_ref[...])

    csum = _dot(llm_probs, tri_ref[...])
    rand = rand_ref[...]
    cf = _dot((csum <= rand).astype(jnp.float32), ones_ref[...])
    cnt = cf.astype(jnp.int32)
    sel = jnp.where(cnt == NL, 0, cnt)
    sel_ref[...] = sel

    n_iota = jax.lax.broadcasted_iota(jnp.int32, llm_probs.shape, 1)
    psel = _dot(jnp.where(n_iota == sel, llm_probs, 0.0), ones_ref[...])
    logp_ref[...] = jnp.log(psel)


@jax.jit
def kernel(enhanced_posts_embeddings, selected_reasoning_embeddings,
           llm_embeddings, gate_W, gate_b, U_W, U_b, V_W, V_b):
    uw = U_W.reshape(RH, 2 * D)
    ub = U_b.reshape(1, RH)
    vw = V_W.reshape(RH, D)
    vbc = V_b.reshape(RH, 1)
    gb2 = gate_b.reshape(1, R)

    cspec = lambda shape: pl.BlockSpec(shape, lambda i: (0,) * len(shape))
    sel, logp, aux = pl.pallas_call(
        _moe_kernel,
        grid=(GRID,),
        in_specs=[
            pl.BlockSpec((TILE, D), lambda i: (i, 0)),
            pl.BlockSpec((TILE, D), lambda i: (i, 0)),
            cspec((NL, D)),
            cspec((R, 2 * D)),
            cspec((1, R)),
            cspec((RH, 2 * D)),
            cspec((1, RH)),
            cspec((RH, D)),
            cspec((RH, 1)),
            cspec((GH, GH)),
            cspec((NL, NL)),
            cspec((GH, NL)),
            cspec((NL, 1)),
            pl.BlockSpec((TILE, 1), lambda i: (i, 0)),
        ],
        out_specs=[
            pl.BlockSpec((TILE, 1), lambda i: (i, 0)),
            pl.BlockSpec((TILE, 1), lambda i: (i, 0)),
            pl.BlockSpec((1, 1), lambda i: (0, 0)),
        ],
        out_shape=[
            jax.ShapeDtypeStruct((B, 1), jnp.int32),
            jax.ShapeDtypeStruct((B, 1), jnp.float32),
            jax.ShapeDtypeStruct((1, 1), jnp.float32),
        ],
        scratch_shapes=[pltpu.VMEM((GH, GH), jnp.float32),
                        pltpu.VMEM((GH, GH), jnp.float32),
                        pltpu.VMEM((2 * D, RH), jnp.float32),
                        pltpu.VMEM((2 * D, R), jnp.float32),
                        pltpu.VMEM((1, R), jnp.float32),
                        pltpu.VMEM((1, R), jnp.float32)],
    )(enhanced_posts_embeddings, selected_reasoning_embeddings,
      llm_embeddings, gate_W, gb2, uw, ub, vw, vbc,
      jnp.asarray(_G4), jnp.asarray(_TRI), jnp.asarray(_F4),
      jnp.asarray(_ONES_COL), jnp.asarray(_RAND))
    return sel[:, 0], logp, aux[0, 0]
